# baseline (speedup 1.0000x reference)
import functools

import jax
import jax.numpy as jnp
from jax import lax
from jax.experimental import pallas as pl
from jax.experimental.pallas import tpu as pltpu
from jax.experimental.pallas import tpu_sc as plsc

K = 20
P = 20

_NC = 2
_NS = 16
_NW = _NC * _NS


def _gather_max_body(k, C, G, M, table_hbm, gidx_hbm, out_hbm,
                     idx_v, rows_v, out_v, sem0, sem1):
    ppw = M // _NW
    n_groups = ppw // G
    WB = 64
    W = WB // G
    wid = lax.axis_index("s") * _NC + lax.axis_index("c")
    pt_base = wid * ppw

    pltpu.sync_copy(
        gidx_hbm.at[pl.ds(pl.multiple_of(wid * n_groups, 8), n_groups)],
        idx_v)

    sems = (sem0, sem1)

    def _start(g, b):
        pltpu.make_async_copy(
            table_hbm.at[idx_v.at[g]], rows_v.at[b], sems[b]).start()

    def _wait(b):
        pltpu.make_async_copy(
            table_hbm.at[idx_v.at[0]], rows_v.at[b], sems[b]).wait()

    _start(0, 0)
    _start(1, 1)

    def outer(i, _):
        for b in range(2):
            g = 2 * i + b
            _wait(b)
            rows_b = rows_v.at[b]
            for p in range(G):
                p_local = (g % W) * G + p
                for cc in range(C // 16):
                    sl = pl.ds(cc * 16, 16)
                    acc = rows_b[p * k, sl]

                    def jbody(j, a):
                        return jnp.maximum(a, rows_b[p * k + j, sl])

                    acc = lax.fori_loop(1, k, jbody, acc, unroll=4)
                    out_v[p_local, sl] = acc

            @pl.when(g + 2 < n_groups)
            def _():
                _start(g + 2, b)

            @pl.when(g % W == W - 1)
            def _():
                off = (g + 1 - W) * G
                pltpu.sync_copy(
                    out_v,
                    out_hbm.at[pl.ds(pl.multiple_of(pt_base + off, 8), WB)])
        return _

    lax.fori_loop(0, n_groups // 2, outer, None)


def _sc_gather_max(table, gidx, k):
    M, C = table.shape
    G = max(1, 120 // k)
    while (M // _NW) % G:
        G -= 1
    n_groups = (M // _NW) // G
    gidx2 = gidx.reshape(M // G, G * k)
    mesh = plsc.VectorSubcoreMesh(core_axis_name="c", subcore_axis_name="s")
    body = functools.partial(_gather_max_body, k, C, G, M)
    fn = pl.kernel(
        body,
        out_type=jax.ShapeDtypeStruct((M, C), jnp.float32),
        mesh=mesh,
        scratch_types=[
            pltpu.VMEM((n_groups, G * k), jnp.int32),
            pltpu.VMEM((2, G * k, C), jnp.float32),
            pltpu.VMEM((64, C), jnp.float32),
            pltpu.SemaphoreType.DMA,
            pltpu.SemaphoreType.DMA,
        ],
        compiler_params=pltpu.CompilerParams(use_tc_tiling_on_sc=False),
    )
    return fn(table, gidx2)


def _gn_lrelu_nc(h, g, b, G):
    N, C = h.shape
    hg = h.reshape(N, G, C // G)
    m = jnp.mean(hg, axis=(0, 2), keepdims=True)
    v = jnp.mean((hg - m) ** 2, axis=(0, 2), keepdims=True)
    hg = (hg - m) * lax.rsqrt(v + 1e-5)
    h = hg.reshape(N, C)
    h = h * g + b
    return jnp.where(h >= 0, h, 0.2 * h)


def _edge_kernel(G, has_res, *refs):
    if has_res:
        (agg_ref, fbin_ref, WaT_ref, ga_ref, ba_ref, WbT_ref, gb_ref, bb_ref,
         fa_ref, fb_ref, xin_ref) = refs
        xin = agg_ref[0] + fbin_ref[0]
        xin_ref[0] = xin
    else:
        (x_ref, WaT_ref, ga_ref, ba_ref, WbT_ref, gb_ref, bb_ref,
         fa_ref, fb_ref) = refs
        xin = x_ref[0]
    fa = jnp.dot(xin, WaT_ref[...], preferred_element_type=jnp.float32)
    fa_ref[0] = _gn_lrelu_nc(fa, ga_ref[...], ba_ref[...], G)
    fb = jnp.dot(xin, WbT_ref[...], preferred_element_type=jnp.float32)
    fb_ref[0] = _gn_lrelu_nc(fb, gb_ref[...], bb_ref[...], G)


def _edge_block(xin_or_aggfb, Wa, ga, ba, Wb, gb, bb, G):
    has_res = len(xin_or_aggfb) == 2
    B, N, Cin = xin_or_aggfb[0].shape
    Cout = Wa.shape[0]
    vec = lambda a: pl.BlockSpec((1, a.shape[0]), lambda b: (0, 0))
    mat = lambda a: pl.BlockSpec(a.shape, lambda b: (0, 0))
    bsp = lambda c: pl.BlockSpec((1, N, c), lambda b: (b, 0, 0))
    in_specs = [bsp(Cin)] * len(xin_or_aggfb) + [
        mat(Wa.T), vec(ga), vec(ba), mat(Wb.T), vec(gb), vec(bb)]
    out_shapes = [jax.ShapeDtypeStruct((B, N, Cout), jnp.float32)] * 2
    out_specs = [bsp(Cout)] * 2
    if has_res:
        out_shapes.append(jax.ShapeDtypeStruct((B, N, Cin), jnp.float32))
        out_specs.append(bsp(Cin))
    outs = pl.pallas_call(
        functools.partial(_edge_kernel, G, has_res),
        grid=(B,),
        in_specs=in_specs,
        out_specs=out_specs,
        out_shape=out_shapes,
    )(*xin_or_aggfb, Wa.T, ga.reshape(1, -1), ba.reshape(1, -1),
      Wb.T, gb.reshape(1, -1), bb.reshape(1, -1))
    if has_res:
        return outs[0], outs[1], outs[2]
    return outs[0], outs[1], xin_or_aggfb[0]


def _l5_kernel(a4_ref, f4_ref, x1_ref, x2_ref, x3_ref,
               WaT_ref, ga_ref, ba_ref, WbT_ref, gb_ref, bb_ref, out_ref):
    x4 = a4_ref[0] + f4_ref[0]
    xc = jnp.concatenate((x1_ref[0], x2_ref[0], x3_ref[0], x4), axis=1)
    h = jnp.dot(xc, WaT_ref[...], preferred_element_type=jnp.float32)
    h = _gn_lrelu_nc(h, ga_ref[...], ba_ref[...], 16)
    h2 = jnp.dot(h, WbT_ref[...], preferred_element_type=jnp.float32)
    out_ref[0] = _gn_lrelu_nc(h2, gb_ref[...], bb_ref[...], 16)


def _layer5(agg4, fb4, x1, x2, x3, W5a, g5a, b5a, W5b, g5b, b5b):
    B, N, _ = agg4.shape
    vec = lambda a: pl.BlockSpec((1, a.shape[0]), lambda b: (0, 0))
    mat = lambda a: pl.BlockSpec(a.shape, lambda b: (0, 0))
    bsp = lambda c: pl.BlockSpec((1, N, c), lambda b: (b, 0, 0))
    return pl.pallas_call(
        _l5_kernel,
        grid=(B,),
        in_specs=[bsp(256), bsp(256), bsp(64), bsp(64), bsp(128),
                  mat(W5a.T), vec(g5a), vec(b5a),
                  mat(W5b.T), vec(g5b), vec(b5b)],
        out_specs=bsp(512),
        out_shape=jax.ShapeDtypeStruct((B, N, 512), jnp.float32),
    )(agg4, fb4, x1, x2, x3, W5a.T, g5a.reshape(1, -1), b5a.reshape(1, -1),
      W5b.T, g5b.reshape(1, -1), b5b.reshape(1, -1))


def _knn_gidx(xt_nc, pool_size):
    B, N, _ = xt_nc.shape
    inner = -2.0 * jnp.einsum('bnc,bmc->bnm', xt_nc, xt_nc)
    xx = jnp.sum(xt_nc ** 2, axis=2)
    pd = -xx[:, :, None] - inner - xx[:, None, :]
    _, idx = jax.lax.top_k(pd, pool_size)
    gidx = idx + (jnp.arange(B, dtype=jnp.int32) * N)[:, None, None]
    return gidx.reshape(B * N, pool_size)


def _point_conv(xin_or_aggfb, Wa, ga, ba, Wb, gb, bb, G, gidx, k):
    fa, fb, xin = _edge_block(xin_or_aggfb, Wa, ga, ba, Wb, gb, bb, G)
    B, N, C = fa.shape
    agg = _sc_gather_max(fa.reshape(B * N, C), gidx[:, :k], k)
    return agg.reshape(B, N, C), fb, xin


def kernel(x, W1a, g1a, b1a, W1b, g1b, b1b, W2a, g2a, b2a, W2b, g2b, b2b, W3a, g3a, b3a, W3b, g3b, b3b, W4a, g4a, b4a, W4b, g4b, b4b, W5a, g5a, b5a, W5b, g5b, b5b):
    pool_size = K + 3 * P
    gidx = _knn_gidx(x, pool_size)
    a1, f1, _ = _point_conv((x,), W1a, g1a, b1a, W1b, g1b, b1b, 8, gidx, K)
    a2, f2, x1 = _point_conv((a1, f1), W2a, g2a, b2a, W2b, g2b, b2b, 8,
                             gidx, K + P)
    a3, f3, x2 = _point_conv((a2, f2), W3a, g3a, b3a, W3b, g3b, b3b, 8,
                             gidx, K + 2 * P)
    a4, f4, x3 = _point_conv((a3, f3), W4a, g4a, b4a, W4b, g4b, b4b, 16,
                             gidx, K + 3 * P)
    return _layer5(a4, f4, x1, x2, x3, W5a, g5a, b5a, W5b, g5b, b5b)


# speedup vs baseline: 1.0957x; 1.0957x over previous
import functools

import jax
import jax.numpy as jnp
from jax import lax
from jax.experimental import pallas as pl
from jax.experimental.pallas import tpu as pltpu
from jax.experimental.pallas import tpu_sc as plsc

K = 20
P = 20

_NC = 2
_NS = 16
_NW = _NC * _NS


def _gather_max_body(k, C, G, M, table_hbm, gidx_hbm, out_hbm,
                     idx_v, rows_v, out_v, sem0, sem1):
    ppw = M // _NW
    n_groups = ppw // G
    WB = 64
    W = WB // G
    wid = lax.axis_index("s") * _NC + lax.axis_index("c")
    pt_base = wid * ppw

    pltpu.sync_copy(
        gidx_hbm.at[pl.ds(pl.multiple_of(wid * n_groups, 8), n_groups)],
        idx_v)

    sems = (sem0, sem1)

    def _start(g, b):
        pltpu.make_async_copy(
            table_hbm.at[idx_v.at[g]], rows_v.at[b], sems[b]).start()

    def _wait(b):
        pltpu.make_async_copy(
            table_hbm.at[idx_v.at[0]], rows_v.at[b], sems[b]).wait()

    _start(0, 0)
    _start(1, 1)

    def outer(i, _):
        for b in range(2):
            g = 2 * i + b
            _wait(b)
            rows_b = rows_v.at[b]
            for p in range(G):
                p_local = (g % W) * G + p
                for cc in range(C // 16):
                    sl = pl.ds(cc * 16, 16)
                    acc = rows_b[p * k, sl]

                    def jbody(j, a):
                        return jnp.maximum(a, rows_b[p * k + j, sl])

                    acc = lax.fori_loop(1, k, jbody, acc, unroll=4)
                    out_v[p_local, sl] = acc

            @pl.when(g + 2 < n_groups)
            def _():
                _start(g + 2, b)

            @pl.when(g % W == W - 1)
            def _():
                off = (g + 1 - W) * G
                pltpu.sync_copy(
                    out_v,
                    out_hbm.at[pl.ds(pl.multiple_of(pt_base + off, 8), WB)])
        return _

    lax.fori_loop(0, n_groups // 2, outer, None)


def _sc_gather_max(table, gidx, k):
    M, C = table.shape
    G = max(1, 120 // k)
    while (M // _NW) % G:
        G -= 1
    n_groups = (M // _NW) // G
    gidx2 = gidx.reshape(M // G, G * k)
    mesh = plsc.VectorSubcoreMesh(core_axis_name="c", subcore_axis_name="s")
    body = functools.partial(_gather_max_body, k, C, G, M)
    fn = pl.kernel(
        body,
        out_type=jax.ShapeDtypeStruct((M, C), jnp.float32),
        mesh=mesh,
        scratch_types=[
            pltpu.VMEM((n_groups, G * k), jnp.int32),
            pltpu.VMEM((2, G * k, C), jnp.float32),
            pltpu.VMEM((64, C), jnp.float32),
            pltpu.SemaphoreType.DMA,
            pltpu.SemaphoreType.DMA,
        ],
        compiler_params=pltpu.CompilerParams(use_tc_tiling_on_sc=False),
    )
    return fn(table, gidx2)


def _gn_lrelu_nc(h, g, b, G):
    N, C = h.shape
    Cg = C // G
    m_c = jnp.mean(h, axis=0, keepdims=True)
    s_c = jnp.mean(h * h, axis=0, keepdims=True)
    mparts, sparts = [], []
    for gi in range(G):
        mg = jnp.mean(m_c[:, gi * Cg:(gi + 1) * Cg])
        sg = jnp.mean(s_c[:, gi * Cg:(gi + 1) * Cg])
        sc = lax.rsqrt(sg - mg * mg + 1e-5)
        mparts.append(jnp.broadcast_to(mg, (1, Cg)))
        sparts.append(jnp.broadcast_to(sc, (1, Cg)))
    mean_c = jnp.concatenate(mparts, axis=1)
    scale_c = jnp.concatenate(sparts, axis=1)
    h = (h - mean_c) * (scale_c * g) + b
    return jnp.where(h >= 0, h, 0.2 * h)


def _edge_kernel(G, has_res, *refs):
    if has_res:
        (agg_ref, fbin_ref, WaT_ref, ga_ref, ba_ref, WbT_ref, gb_ref, bb_ref,
         fa_ref, fb_ref, xin_ref) = refs
        xin = agg_ref[0] + fbin_ref[0]
        xin_ref[0] = xin
    else:
        (x_ref, WaT_ref, ga_ref, ba_ref, WbT_ref, gb_ref, bb_ref,
         fa_ref, fb_ref) = refs
        xin = x_ref[0]
    fa = jnp.dot(xin, WaT_ref[...], preferred_element_type=jnp.float32)
    fa_ref[0] = _gn_lrelu_nc(fa, ga_ref[...], ba_ref[...], G)
    fb = jnp.dot(xin, WbT_ref[...], preferred_element_type=jnp.float32)
    fb_ref[0] = _gn_lrelu_nc(fb, gb_ref[...], bb_ref[...], G)


def _edge_block(xin_or_aggfb, Wa, ga, ba, Wb, gb, bb, G):
    has_res = len(xin_or_aggfb) == 2
    B, N, Cin = xin_or_aggfb[0].shape
    Cout = Wa.shape[0]
    vec = lambda a: pl.BlockSpec((1, a.shape[0]), lambda b: (0, 0))
    mat = lambda a: pl.BlockSpec(a.shape, lambda b: (0, 0))
    bsp = lambda c: pl.BlockSpec((1, N, c), lambda b: (b, 0, 0))
    in_specs = [bsp(Cin)] * len(xin_or_aggfb) + [
        mat(Wa.T), vec(ga), vec(ba), mat(Wb.T), vec(gb), vec(bb)]
    out_shapes = [jax.ShapeDtypeStruct((B, N, Cout), jnp.float32)] * 2
    out_specs = [bsp(Cout)] * 2
    if has_res:
        out_shapes.append(jax.ShapeDtypeStruct((B, N, Cin), jnp.float32))
        out_specs.append(bsp(Cin))
    outs = pl.pallas_call(
        functools.partial(_edge_kernel, G, has_res),
        grid=(B,),
        in_specs=in_specs,
        out_specs=out_specs,
        out_shape=out_shapes,
    )(*xin_or_aggfb, Wa.T, ga.reshape(1, -1), ba.reshape(1, -1),
      Wb.T, gb.reshape(1, -1), bb.reshape(1, -1))
    if has_res:
        return outs[0], outs[1], outs[2]
    return outs[0], outs[1], xin_or_aggfb[0]


def _l5_kernel(a4_ref, f4_ref, x1_ref, x2_ref, x3_ref,
               WaT_ref, ga_ref, ba_ref, WbT_ref, gb_ref, bb_ref, out_ref):
    x4 = a4_ref[0] + f4_ref[0]
    xc = jnp.concatenate((x1_ref[0], x2_ref[0], x3_ref[0], x4), axis=1)
    h = jnp.dot(xc, WaT_ref[...], preferred_element_type=jnp.float32)
    h = _gn_lrelu_nc(h, ga_ref[...], ba_ref[...], 16)
    h2 = jnp.dot(h, WbT_ref[...], preferred_element_type=jnp.float32)
    out_ref[0] = _gn_lrelu_nc(h2, gb_ref[...], bb_ref[...], 16)


def _layer5(agg4, fb4, x1, x2, x3, W5a, g5a, b5a, W5b, g5b, b5b):
    B, N, _ = agg4.shape
    vec = lambda a: pl.BlockSpec((1, a.shape[0]), lambda b: (0, 0))
    mat = lambda a: pl.BlockSpec(a.shape, lambda b: (0, 0))
    bsp = lambda c: pl.BlockSpec((1, N, c), lambda b: (b, 0, 0))
    return pl.pallas_call(
        _l5_kernel,
        grid=(B,),
        in_specs=[bsp(256), bsp(256), bsp(64), bsp(64), bsp(128),
                  mat(W5a.T), vec(g5a), vec(b5a),
                  mat(W5b.T), vec(g5b), vec(b5b)],
        out_specs=bsp(512),
        out_shape=jax.ShapeDtypeStruct((B, N, 512), jnp.float32),
    )(agg4, fb4, x1, x2, x3, W5a.T, g5a.reshape(1, -1), b5a.reshape(1, -1),
      W5b.T, g5b.reshape(1, -1), b5b.reshape(1, -1))


_TK = 80


def _knn_gidx(xt_nc, pool_size):
    B, N, _ = xt_nc.shape
    inner = -2.0 * jnp.einsum('bnc,bmc->bnm', xt_nc, xt_nc)
    xx = jnp.sum(xt_nc ** 2, axis=2)
    pd = -xx[:, :, None] - inner - xx[:, None, :]
    _, idx = jax.lax.top_k(pd, pool_size)
    gidx = idx + (jnp.arange(B, dtype=jnp.int32) * N)[:, None, None]
    return gidx.reshape(B * N, pool_size)


def _point_conv(xin_or_aggfb, Wa, ga, ba, Wb, gb, bb, G, gidx, k):
    fa, fb, xin = _edge_block(xin_or_aggfb, Wa, ga, ba, Wb, gb, bb, G)
    B, N, C = fa.shape
    agg = _sc_gather_max(fa.reshape(B * N, C), gidx[:, :k], k)
    return agg.reshape(B, N, C), fb, xin


def kernel(x, W1a, g1a, b1a, W1b, g1b, b1b, W2a, g2a, b2a, W2b, g2b, b2b, W3a, g3a, b3a, W3b, g3b, b3b, W4a, g4a, b4a, W4b, g4b, b4b, W5a, g5a, b5a, W5b, g5b, b5b):
    pool_size = K + 3 * P
    gidx = _knn_gidx(x, pool_size)
    a1, f1, _ = _point_conv((x,), W1a, g1a, b1a, W1b, g1b, b1b, 8, gidx, K)
    a2, f2, x1 = _point_conv((a1, f1), W2a, g2a, b2a, W2b, g2b, b2b, 8,
                             gidx, K + P)
    a3, f3, x2 = _point_conv((a2, f2), W3a, g3a, b3a, W3b, g3b, b3b, 8,
                             gidx, K + 2 * P)
    a4, f4, x3 = _point_conv((a3, f3), W4a, g4a, b4a, W4b, g4b, b4b, 16,
                             gidx, K + 3 * P)
    return _layer5(a4, f4, x1, x2, x3, W5a, g5a, b5a, W5b, g5b, b5b)
